# baseline (speedup 1.0000x reference)
import jax
import jax.numpy as jnp
from jax.experimental import pallas as pl
from jax.experimental.pallas import tpu as pltpu


def _vq_block_kernel(z_ref, emb_ref, embt_ref, zq_ref, idx_ref, loss_ref,
                     e2_ref):
    zb = z_ref[0]
    k_dim = emb_ref.shape[0]
    t_blk = zb.shape[1]
    first = jnp.logical_and(pl.program_id(0) == 0, pl.program_id(1) == 0)

    @pl.when(first)
    def _init():
        emb = emb_ref[...]
        e2_ref[...] = jnp.sum(emb * emb, axis=1, keepdims=True)
        loss_ref[...] = jnp.zeros((1, 1), jnp.float32)

    scores = jnp.dot(emb_ref[...], zb, preferred_element_type=jnp.float32)
    dist = e2_ref[...] - 2.0 * scores
    idx = jnp.argmin(dist, axis=0)

    onehot = (jax.lax.broadcasted_iota(jnp.int32, (k_dim, t_blk), 0)
              == idx[None, :]).astype(jnp.float32)
    zq = jnp.dot(embt_ref[...], onehot,
                 preferred_element_type=jnp.float32)
    zq_ref[0] = zq
    idx_ref[0, 0] = idx

    resid = zb - zq
    loss_ref[...] += jnp.sum(resid * resid, axis=(0, 1),
                             keepdims=True)


@jax.jit
def kernel(z, embedding):
    B, D, T = z.shape
    K = embedding.shape[0]
    t_blk = 512
    nt = T // t_blk

    grid = (B, nt)
    zq, idx3, loss_raw = pl.pallas_call(
        _vq_block_kernel,
        grid=grid,
        in_specs=[
            pl.BlockSpec((1, D, t_blk), lambda b, t: (b, 0, t)),
            pl.BlockSpec((K, D), lambda b, t: (0, 0)),
            pl.BlockSpec((D, K), lambda b, t: (0, 0)),
        ],
        out_specs=[
            pl.BlockSpec((1, D, t_blk), lambda b, t: (b, 0, t)),
            pl.BlockSpec((1, 1, t_blk), lambda b, t: (b * nt + t, 0, 0)),
            pl.BlockSpec((1, 1), lambda b, t: (0, 0)),
        ],
        out_shape=[
            jax.ShapeDtypeStruct((B, D, T), jnp.float32),
            jax.ShapeDtypeStruct((B * nt, 1, t_blk), jnp.int32),
            jax.ShapeDtypeStruct((1, 1), jnp.float32),
        ],
        scratch_shapes=[pltpu.VMEM((K, 1), jnp.float32)],
        compiler_params=pltpu.CompilerParams(
            dimension_semantics=("arbitrary", "arbitrary"),
        ),
    )(z, embedding, embedding.T)

    loss = loss_raw[0, 0] * (0.25 / (B * T * D))
    indices = idx3.reshape(B, T)
    return zq, loss, indices


# speedup vs baseline: 1.0147x; 1.0147x over previous
import jax
import jax.numpy as jnp
from jax.experimental import pallas as pl
from jax.experimental.pallas import tpu as pltpu


def _vq_block_kernel(z_ref, emb_ref, embt_ref, zq_ref, idx_ref, loss_ref,
                     e2_ref):
    k_dim = emb_ref.shape[0]
    t_blk = z_ref.shape[2]
    half = t_blk // 2
    first = jnp.logical_and(pl.program_id(0) == 0, pl.program_id(1) == 0)

    @pl.when(first)
    def _init():
        emb = emb_ref[...]
        e2_ref[...] = jnp.sum(emb * emb, axis=1, keepdims=True)
        loss_ref[...] = jnp.zeros((1, 1), jnp.float32)

    emb = emb_ref[...]
    embt = embt_ref[...]
    e2 = e2_ref[...]
    iota_k = jax.lax.broadcasted_iota(jnp.int32, (k_dim, half), 0)

    def _half(zb):
        scores = jnp.dot(emb, zb, preferred_element_type=jnp.float32)
        dist = e2 - 2.0 * scores
        idx = jnp.argmin(dist, axis=0)
        onehot = (iota_k == idx[None, :]).astype(jnp.bfloat16)
        zq = jnp.dot(embt, onehot,
                     preferred_element_type=jnp.float32)
        resid = zb - zq
        part = jnp.sum(resid * resid, axis=(0, 1), keepdims=True)
        return zq, idx, part

    zq0, idx0, part0 = _half(z_ref[0, :, :half])
    zq1, idx1, part1 = _half(z_ref[0, :, half:])

    zq_ref[0, :, :half] = zq0
    zq_ref[0, :, half:] = zq1
    idx_ref[0, 0, :half] = idx0
    idx_ref[0, 0, half:] = idx1
    loss_ref[...] += part0 + part1


@jax.jit
def kernel(z, embedding):
    B, D, T = z.shape
    K = embedding.shape[0]
    t_blk = 512
    nt = T // t_blk

    grid = (B, nt)
    zq, idx3, loss_raw = pl.pallas_call(
        _vq_block_kernel,
        grid=grid,
        in_specs=[
            pl.BlockSpec((1, D, t_blk), lambda b, t: (b, 0, t)),
            pl.BlockSpec((K, D), lambda b, t: (0, 0)),
            pl.BlockSpec((D, K), lambda b, t: (0, 0)),
        ],
        out_specs=[
            pl.BlockSpec((1, D, t_blk), lambda b, t: (b, 0, t)),
            pl.BlockSpec((1, 1, t_blk), lambda b, t: (b * nt + t, 0, 0)),
            pl.BlockSpec((1, 1), lambda b, t: (0, 0)),
        ],
        out_shape=[
            jax.ShapeDtypeStruct((B, D, T), jnp.float32),
            jax.ShapeDtypeStruct((B * nt, 1, t_blk), jnp.int32),
            jax.ShapeDtypeStruct((1, 1), jnp.float32),
        ],
        scratch_shapes=[pltpu.VMEM((K, 1), jnp.float32)],
        compiler_params=pltpu.CompilerParams(
            dimension_semantics=("arbitrary", "arbitrary"),
        ),
    )(z, embedding, embedding.T.astype(jnp.bfloat16))

    loss = loss_raw[0, 0] * (0.25 / (B * T * D))
    indices = idx3.reshape(B, T)
    return zq, loss, indices


# speedup vs baseline: 1.1777x; 1.1607x over previous
import jax
import jax.numpy as jnp
from jax.experimental import pallas as pl
from jax.experimental.pallas import tpu as pltpu


def _vq_block_kernel(zm_ref, emb_ref, embt_ref, zq_ref, idx_ref, loss_ref,
                     e2_ref):
    k_dim = emb_ref.shape[0]
    t_blk = zm_ref.shape[2]
    half = t_blk // 2

    @pl.when(pl.program_id(0) == 0)
    def _init():
        emb = emb_ref[...]
        e2_ref[...] = 0.25 * jnp.sum(emb * emb, axis=1, keepdims=True)
        loss_ref[...] = jnp.zeros((1, 1), jnp.float32)

    emb = emb_ref[...]
    embt = embt_ref[...]
    e2 = e2_ref[...]
    iota_k = jax.lax.broadcasted_iota(jnp.int32, (k_dim, half), 0)

    def _half(zb):
        scores = jnp.dot(emb, zb, preferred_element_type=jnp.float32)
        dist = e2 + scores
        idx = jnp.argmin(dist, axis=0)
        onehot = (iota_k == idx[None, :]).astype(jnp.bfloat16)
        zq = jnp.dot(embt, onehot,
                     preferred_element_type=jnp.float32)
        resid = zb - zq
        part = jnp.sum(resid * resid, axis=(0, 1), keepdims=True)
        return zq, idx, part

    zq0, idx0, part0 = _half(zm_ref[0, :, :half])
    zq1, idx1, part1 = _half(zm_ref[0, :, half:])

    zq_ref[0, :, :half] = zq0
    zq_ref[0, :, half:] = zq1
    idx_ref[0, 0, :half] = idx0
    idx_ref[0, 0, half:] = idx1
    loss_ref[...] += part0 + part1


@jax.jit
def kernel(z, embedding):
    B, D, T = z.shape
    K = embedding.shape[0]

    grid = (B,)
    zq, idx3, loss_raw = pl.pallas_call(
        _vq_block_kernel,
        grid=grid,
        in_specs=[
            pl.BlockSpec((1, D, T), lambda b: (b, 0, 0)),
            pl.BlockSpec((K, D), lambda b: (0, 0)),
            pl.BlockSpec((D, K), lambda b: (0, 0)),
        ],
        out_specs=[
            pl.BlockSpec((1, D, T), lambda b: (b, 0, 0)),
            pl.BlockSpec((1, 1, T), lambda b: (b, 0, 0)),
            pl.BlockSpec((1, 1), lambda b: (0, 0)),
        ],
        out_shape=[
            jax.ShapeDtypeStruct((B, D, T), jnp.float32),
            jax.ShapeDtypeStruct((B, 1, T), jnp.int32),
            jax.ShapeDtypeStruct((1, 1), jnp.float32),
        ],
        scratch_shapes=[pltpu.VMEM((K, 1), jnp.float32)],
        compiler_params=pltpu.CompilerParams(
            dimension_semantics=("arbitrary",),
        ),
    )(z, embedding * (-2.0), embedding.T.astype(jnp.bfloat16))

    loss = loss_raw[0, 0] * (0.25 / (B * T * D))
    indices = idx3.reshape(B, T)
    return zq, loss, indices
